# baseline (speedup 1.0000x reference)
import functools

import jax
import jax.numpy as jnp
from jax import lax
from jax.experimental import pallas as pl
from jax.experimental.pallas import tpu as pltpu
from jax.experimental.pallas import tpu_sc as plsc

NC = 2
NS = 16
NW = NC * NS
L = 16
CHUNK = 16
LN_EPS = 1e-12


def _rsqrt(x):
    i = lax.bitcast_convert_type(x, jnp.int32)
    i = jnp.full(i.shape, 0x5F3759DF, jnp.int32) - lax.shift_right_logical(i, 1)
    y = lax.bitcast_convert_type(i, jnp.float32)
    y = y * (1.5 - 0.5 * x * y * y)
    y = y * (1.5 - 0.5 * x * y * y)
    y = y * (1.5 - 0.5 * x * y * y)
    return y


_GATHER_DNUMS = lax.GatherDimensionNumbers(
    offset_dims=(), collapsed_slice_dims=(0,), start_index_map=(0,))


def _rotate(x, sh):
    perm = lax.bitwise_and(lax.iota(jnp.int32, L) + sh, L - 1)
    return lax.gather(x, perm[:, None], _GATHER_DNUMS, (1,),
                      mode=lax.GatherScatterMode.PROMISE_IN_BOUNDS)


def _lane_total(x):
    for sh in (8, 4, 2, 1):
        x = x + _rotate(x, sh)
    return x


def _make_sc_kernel(n_rows, seq_len, d):
    ng = d // L
    rows_per_w = n_rows // NW
    n_chunks = rows_per_w // CHUNK

    def body(idx_hbm, word_hbm, pe_hbm, tt_hbm, gamma_hbm, beta_hbm, out_hbm,
             idx_v, rows_v, pe_v, tt_v, gamma_v, beta_v, sem):
        wid = lax.axis_index("s") * NC + lax.axis_index("c")
        base_row = wid * rows_per_w

        pltpu.sync_copy(idx_hbm.at[pl.ds(wid * n_chunks, n_chunks)], idx_v)
        pltpu.sync_copy(pe_hbm, pe_v)
        pltpu.sync_copy(tt_hbm.at[pl.ds(0, 1)], tt_v)
        pltpu.sync_copy(gamma_hbm, gamma_v)
        pltpu.sync_copy(beta_hbm, beta_v)

        def fold(i, _):
            p = i // ng
            g = i % ng
            sl = pl.ds(g * L, L)
            pe_v[p, sl] = pe_v[p, sl] + tt_v[0, sl]
            return 0
        lax.fori_loop(0, seq_len * ng, fold, 0)

        def chunk_body(k, _):
            row0 = base_row + k * CHUNK
            pltpu.async_copy(word_hbm.at[idx_v.at[k]], rows_v, sem).wait()

            def row_body(s, _):
                pos = lax.rem(row0 + s, seq_len)

                def grp_a(g, carry):
                    acc, acc2 = carry
                    sl = pl.ds(g * L, L)
                    h = rows_v[s, sl] + pe_v[pos, sl]
                    rows_v[s, sl] = h
                    return acc + h, acc2 + h * h

                acc, acc2 = lax.fori_loop(
                    0, ng, grp_a,
                    (jnp.zeros((L,), jnp.float32), jnp.zeros((L,), jnp.float32)))
                mu = _lane_total(acc) * (1.0 / d)
                var = _lane_total(acc2) * (1.0 / d) - mu * mu
                rs = _rsqrt(var + LN_EPS)

                def grp_b(g, _):
                    sl = pl.ds(g * L, L)
                    o = (rows_v[s, sl] - mu) * rs
                    rows_v[s, sl] = o * gamma_v[sl] + beta_v[sl]
                    return 0
                lax.fori_loop(0, ng, grp_b, 0)
                return 0

            lax.fori_loop(0, CHUNK, row_body, 0)
            pltpu.sync_copy(rows_v, out_hbm.at[pl.ds(row0, CHUNK)])
            return 0

        lax.fori_loop(0, n_chunks, chunk_body, 0)

    return pl.kernel(
        body,
        out_type=jax.ShapeDtypeStruct((n_rows, d), jnp.float32),
        mesh=plsc.VectorSubcoreMesh(core_axis_name="c", subcore_axis_name="s"),
        scratch_types=[
            pltpu.VMEM((n_chunks, CHUNK), jnp.int32),
            pltpu.VMEM((CHUNK, d), jnp.float32),
            pltpu.VMEM((seq_len, d), jnp.float32),
            pltpu.VMEM((1, d), jnp.float32),
            pltpu.VMEM((d,), jnp.float32),
            pltpu.VMEM((d,), jnp.float32),
            pltpu.SemaphoreType.DMA,
        ],
    )


@jax.jit
def _run(x_i32, word_emb, pos_emb, tt_emb, ln_gamma, ln_beta):
    b, seq_len = x_i32.shape
    d = word_emb.shape[1]
    n_rows = b * seq_len
    idx2 = x_i32.reshape(n_rows // CHUNK, CHUNK)
    sc = _make_sc_kernel(n_rows, seq_len, d)
    out = sc(idx2, word_emb, pos_emb, tt_emb, ln_gamma, ln_beta)
    return out.reshape(b, seq_len, d)


def kernel(x, word_emb, pos_emb, tt_emb, ln_gamma, ln_beta):
    return _run(x.astype(jnp.int32), word_emb, pos_emb, tt_emb,
                ln_gamma, ln_beta)


# speedup vs baseline: 1.3135x; 1.3135x over previous
import functools

import jax
import jax.numpy as jnp
from jax import lax
from jax.experimental import pallas as pl
from jax.experimental.pallas import tpu as pltpu
from jax.experimental.pallas import tpu_sc as plsc

NC = 2
NS = 16
NW = NC * NS
L = 16
LQ = 4
BG = NW // LQ
CHUNK = 32
LN_EPS = 1e-12


def _rsqrt(x):
    i = lax.bitcast_convert_type(x, jnp.int32)
    i = jnp.full(i.shape, 0x5F3759DF, jnp.int32) - lax.shift_right_logical(i, 1)
    y = lax.bitcast_convert_type(i, jnp.float32)
    y = y * (1.5 - 0.5 * x * y * y)
    y = y * (1.5 - 0.5 * x * y * y)
    y = y * (1.5 - 0.5 * x * y * y)
    return y


_GATHER_DNUMS = lax.GatherDimensionNumbers(
    offset_dims=(), collapsed_slice_dims=(0,), start_index_map=(0,))


def _rotate(x, sh):
    perm = lax.bitwise_and(lax.iota(jnp.int32, L) + sh, L - 1)
    return lax.gather(x, perm[:, None], _GATHER_DNUMS, (1,),
                      mode=lax.GatherScatterMode.PROMISE_IN_BOUNDS)


def _lane_total(x):
    for sh in (8, 4, 2, 1):
        x = x + _rotate(x, sh)
    return x


def _fold_tt(pe_ref, tt_ref, o_ref):
    o_ref[...] = pe_ref[...] + tt_ref[0][None, None, :]


def _make_sc_kernel(n_seq, seq_len, d):
    ng = d // L
    pos_per_w = seq_len // LQ
    rows_per_w = (n_seq // BG) * pos_per_w
    n_chunks = rows_per_w // CHUNK

    def body(idx_hbm, oidx_hbm, word_hbm, pe_hbm, gamma_hbm, beta_hbm,
             out_hbm, idx_v, oidx_v, rows_v, pe_v, gamma_v, beta_v, gsem):
        wid = lax.axis_index("s") * NC + lax.axis_index("c")
        lg = lax.rem(wid, LQ)

        pltpu.sync_copy(idx_hbm.at[wid], idx_v)
        pltpu.sync_copy(oidx_hbm.at[wid], oidx_v)
        pltpu.sync_copy(pe_hbm.at[lg], pe_v)
        pltpu.sync_copy(gamma_hbm, gamma_v)
        pltpu.sync_copy(beta_hbm, beta_v)

        def chunk_body(j, _):
            pltpu.async_copy(word_hbm.at[idx_v.at[j]], rows_v, gsem).wait()

            def row_body(s, _):
                pos = lax.rem(j * CHUNK + s, pos_per_w)
                acc = None
                acc2 = None
                for g in range(ng):
                    sl = pl.ds(g * L, L)
                    h = rows_v[s, sl] + pe_v[pos, sl]
                    rows_v[s, sl] = h
                    acc = h if acc is None else acc + h
                    acc2 = h * h if acc2 is None else acc2 + h * h
                mu = _lane_total(acc) * (1.0 / d)
                var = _lane_total(acc2) * (1.0 / d) - mu * mu
                rs = _rsqrt(var + LN_EPS)
                for g in range(ng):
                    sl = pl.ds(g * L, L)
                    o = (rows_v[s, sl] - mu) * rs
                    rows_v[s, sl] = o * gamma_v[sl] + beta_v[sl]
                return 0

            lax.fori_loop(0, CHUNK, row_body, 0)
            pltpu.async_copy(rows_v, out_hbm.at[oidx_v.at[j]], gsem).wait()
            return 0

        lax.fori_loop(0, n_chunks, chunk_body, 0)

    return pl.kernel(
        body,
        out_type=jax.ShapeDtypeStruct((n_seq * seq_len, d), jnp.float32),
        mesh=plsc.VectorSubcoreMesh(core_axis_name="c", subcore_axis_name="s"),
        scratch_types=[
            pltpu.VMEM((n_chunks, CHUNK), jnp.int32),
            pltpu.VMEM((n_chunks, CHUNK), jnp.int32),
            pltpu.VMEM((CHUNK, d), jnp.float32),
            pltpu.VMEM((pos_per_w, d), jnp.float32),
            pltpu.VMEM((d,), jnp.float32),
            pltpu.VMEM((d,), jnp.float32),
            pltpu.SemaphoreType.DMA,
        ],
    )


@jax.jit
def _run(x_i32, word_emb, pos_emb, tt_emb, ln_gamma, ln_beta):
    n_seq, seq_len = x_i32.shape
    d = word_emb.shape[1]
    pos_per_w = seq_len // LQ
    seq_per_w = n_seq // BG
    rows_per_w = seq_per_w * pos_per_w

    idx_t = jnp.transpose(
        x_i32.reshape(BG, seq_per_w, LQ, pos_per_w), (0, 2, 1, 3)
    ).reshape(NW, rows_per_w // CHUNK, CHUNK)

    bgs = jnp.arange(BG, dtype=jnp.int32)[:, None, None, None]
    lgs = jnp.arange(LQ, dtype=jnp.int32)[None, :, None, None]
    bis = jnp.arange(seq_per_w, dtype=jnp.int32)[None, None, :, None]
    ps = jnp.arange(pos_per_w, dtype=jnp.int32)[None, None, None, :]
    oidx = ((bgs * seq_per_w + bis) * seq_len + lgs * pos_per_w + ps)
    oidx = jnp.broadcast_to(oidx, (BG, LQ, seq_per_w, pos_per_w)).reshape(
        NW, rows_per_w // CHUNK, CHUNK)

    pe_tt = pl.pallas_call(
        _fold_tt,
        out_shape=jax.ShapeDtypeStruct((LQ, pos_per_w, d), jnp.float32),
    )(pos_emb.reshape(LQ, pos_per_w, d), tt_emb)
    sc = _make_sc_kernel(n_seq, seq_len, d)
    out = sc(idx_t, oidx, word_emb, pe_tt, ln_gamma, ln_beta)
    return out.reshape(n_seq, seq_len, d)


def kernel(x, word_emb, pos_emb, tt_emb, ln_gamma, ln_beta):
    return _run(x.astype(jnp.int32), word_emb, pos_emb, tt_emb,
                ln_gamma, ln_beta)
